# baseline (speedup 1.0000x reference)
import functools

import jax
import jax.numpy as jnp
from jax import lax
from jax.experimental import pallas as pl
from jax.experimental.pallas import tpu as pltpu
from jax.experimental.pallas import tpu_sc as plsc

_N = 10000
_E = 320000
_D = 128
_C = 10

_NC = 2
_NS = 16
_NW = _NC * _NS
_EPW = _E // _NW
_NP = 10240
_RPT = _NP // _NS

_sc_mesh = functools.partial(
    plsc.VectorSubcoreMesh, core_axis_name="c", subcore_axis_name="s")
_sc_params = pltpu.CompilerParams(use_tc_tiling_on_sc=False)


def _make_sc_agg(width, K):
  nchunks = _EPW // K

  @functools.partial(
      pl.kernel,
      mesh=_sc_mesh(),
      out_type=jax.ShapeDtypeStruct((_NC, _NP, width), jnp.float32),
      compiler_params=_sc_params,
      scratch_types=[
          pltpu.VMEM((nchunks, K), jnp.int32),
          pltpu.VMEM((nchunks, K), jnp.int32),
          pltpu.VMEM((K, width), jnp.float32),
          pltpu.VMEM((K, width), jnp.float32),
          pltpu.VMEM_SHARED((_NP, width), jnp.float32),
          pltpu.SemaphoreType.DMA,
          pltpu.SemaphoreType.DMA,
          pltpu.SemaphoreType.DMA,
          pltpu.SemaphoreType.DMA,
          pltpu.SemaphoreType.DMA,
      ],
  )
  def agg(table_hbm, src_hbm, dst_hbm, zeros_hbm, out_hbm,
          srcb, dstb, rows0, rows1, acc, sem0, sem1, sem2, sem3, semi):
    c = lax.axis_index("c")
    s = lax.axis_index("s")
    wid = s * _NC + c
    idx_load = pltpu.async_copy(src_hbm.at[wid], srcb, semi)
    idx_load2 = pltpu.async_copy(dst_hbm.at[wid], dstb, semi)
    pltpu.sync_copy(zeros_hbm, acc.at[pl.ds(s * _RPT, _RPT)])
    idx_load.wait()
    idx_load2.wait()
    plsc.subcore_barrier()

    rows = (rows0, rows1)
    gsems = (sem0, sem1)
    ssems = (sem2, sem3)
    gd = [None, None]
    sd = [None, None]
    gd[0] = pltpu.async_copy(table_hbm.at[srcb.at[0]], rows0, gsems[0])
    for i in range(nchunks):
      cur = i % 2
      nxt = 1 - cur
      if i + 1 < nchunks:
        if sd[nxt] is not None:
          sd[nxt].wait()
        gd[nxt] = pltpu.async_copy(table_hbm.at[srcb.at[i + 1]], rows[nxt],
                                   gsems[nxt])
      gd[cur].wait()
      sd[cur] = pltpu.async_copy(rows[cur], acc.at[dstb.at[i]], ssems[cur],
                                 add=True)
    sd[(nchunks - 1) % 2].wait()
    if nchunks > 1:
      sd[nchunks % 2].wait()
    plsc.subcore_barrier()
    pltpu.sync_copy(acc.at[pl.ds(s * _RPT, _RPT)],
                    out_hbm.at[c, pl.ds(s * _RPT, _RPT)])

  return agg


def _make_sc_deg(K):
  nchunks = _EPW // K

  @functools.partial(
      pl.kernel,
      mesh=_sc_mesh(),
      out_type=jax.ShapeDtypeStruct((_NC, _NP, 16), jnp.float32),
      compiler_params=_sc_params,
      scratch_types=[
          pltpu.VMEM((nchunks, K), jnp.int32),
          pltpu.VMEM((K, 16), jnp.float32),
          pltpu.VMEM_SHARED((_NP, 16), jnp.float32),
          pltpu.SemaphoreType.DMA,
      ],
  )
  def deg(dst_hbm, ones_hbm, zeros_hbm, out_hbm, dstb, ones_v, acc, semi):
    c = lax.axis_index("c")
    s = lax.axis_index("s")
    wid = s * _NC + c
    idx_load = pltpu.async_copy(dst_hbm.at[wid], dstb, semi)
    pltpu.sync_copy(zeros_hbm, acc.at[pl.ds(s * _RPT, _RPT)])
    pltpu.sync_copy(ones_hbm, ones_v)
    idx_load.wait()
    plsc.subcore_barrier()

    for i in range(nchunks):
      pltpu.sync_copy(ones_v, acc.at[dstb.at[i]], add=True)
    plsc.subcore_barrier()
    pltpu.sync_copy(acc.at[pl.ds(s * _RPT, _RPT)],
                    out_hbm.at[c, pl.ds(s * _RPT, _RPT)])

  return deg


_BLK = 1000
_GRID = _N // _BLK


def _dinv_of(degp_blk):
  deg = degp_blk[0, :, 0:1] + degp_blk[1, :, 0:1] + 1.0
  return lax.rsqrt(deg)


def _tc1m_body(x_ref, w1_ref, h_ref):
  h_ref[...] = jnp.dot(x_ref[...], w1_ref[...],
                       preferred_element_type=jnp.float32)


@jax.jit
def _tc1m(x, w1):
  return pl.pallas_call(
      _tc1m_body,
      grid=(_GRID,),
      in_specs=[
          pl.BlockSpec((_BLK, _D), lambda i: (i, 0)),
          pl.BlockSpec((_D, _D), lambda i: (0, 0)),
      ],
      out_specs=pl.BlockSpec((_BLK, _D), lambda i: (i, 0)),
      out_shape=jax.ShapeDtypeStruct((_N, _D), jnp.float32),
  )(x, w1)


def _tc1s_body(h_ref, degp_ref, hs_ref):
  hs_ref[...] = h_ref[...] * _dinv_of(degp_ref[...])


@jax.jit
def _tc1s(h, degp):
  return pl.pallas_call(
      _tc1s_body,
      grid=(_GRID,),
      in_specs=[
          pl.BlockSpec((_BLK, _D), lambda i: (i, 0)),
          pl.BlockSpec((_NC, _BLK, 16), lambda i: (0, i, 0)),
      ],
      out_specs=pl.BlockSpec((_BLK, _D), lambda i: (i, 0)),
      out_shape=jax.ShapeDtypeStruct((_N, _D), jnp.float32),
  )(h, degp)


def _tc2_body(agg_ref, hs_ref, degp_ref, w2_ref, b1_ref, ps_ref):
  dinv = _dinv_of(degp_ref[...])
  agg = agg_ref[0] + agg_ref[1] + hs_ref[...]
  h = jnp.maximum(agg * dinv + b1_ref[...], 0.0)
  p = jnp.dot(h, w2_ref[...], preferred_element_type=jnp.float32)
  ps_ref[...] = p * dinv


@jax.jit
def _tc2(agg1, hs, degp, w2p, b1):
  return pl.pallas_call(
      _tc2_body,
      grid=(_GRID,),
      in_specs=[
          pl.BlockSpec((_NC, _BLK, _D), lambda i: (0, i, 0)),
          pl.BlockSpec((_BLK, _D), lambda i: (i, 0)),
          pl.BlockSpec((_NC, _BLK, 16), lambda i: (0, i, 0)),
          pl.BlockSpec((_D, 16), lambda i: (0, 0)),
          pl.BlockSpec((1, _D), lambda i: (0, 0)),
      ],
      out_specs=pl.BlockSpec((_BLK, 16), lambda i: (i, 0)),
      out_shape=jax.ShapeDtypeStruct((_N, 16), jnp.float32),
  )(agg1, hs, degp, w2p, b1)


def _tc3_body(agg_ref, ps_ref, degp_ref, b2_ref, out_ref):
  dinv = _dinv_of(degp_ref[...])
  z = (agg_ref[0] + agg_ref[1] + ps_ref[...]) * dinv + b2_ref[...]
  m = jnp.max(z, axis=1, keepdims=True)
  e = jnp.exp(z - m)
  out_ref[...] = e / jnp.sum(e, axis=1, keepdims=True)


@jax.jit
def _tc3(agg2, ps, degp, b2p):
  return pl.pallas_call(
      _tc3_body,
      grid=(_GRID,),
      in_specs=[
          pl.BlockSpec((_NC, _BLK, 16), lambda i: (0, i, 0)),
          pl.BlockSpec((_BLK, 16), lambda i: (i, 0)),
          pl.BlockSpec((_NC, _BLK, 16), lambda i: (0, i, 0)),
          pl.BlockSpec((1, 16), lambda i: (0, 0)),
      ],
      out_specs=pl.BlockSpec((_BLK, 16), lambda i: (i, 0)),
      out_shape=jax.ShapeDtypeStruct((_N, 16), jnp.float32),
  )(agg2, ps, degp, b2p)


_sc_deg = _make_sc_deg(K=2000)
_sc_agg128 = _make_sc_agg(width=_D, K=80)
_sc_agg16 = _make_sc_agg(width=16, K=2000)


@jax.jit
def kernel(node_features, edge_index, W1, b1, W2, b2):
  src = edge_index[0]
  dst = edge_index[1]
  src_a = src.reshape(_NW, _EPW // 80, 80)
  dst_a = dst.reshape(_NW, _EPW // 80, 80)
  src_b = src.reshape(_NW, _EPW // 2000, 2000)
  dst_b = dst.reshape(_NW, _EPW // 2000, 2000)
  f32 = jnp.float32
  zeros16 = jnp.zeros((_RPT, 16), f32)
  zeros128 = jnp.zeros((_RPT, _D), f32)
  ones_rows = jnp.ones((2000, 16), f32)

  h1 = _tc1m(node_features, W1)
  degp = _sc_deg(dst_b, ones_rows, zeros16)
  hs1 = _tc1s(h1, degp)
  agg1 = _sc_agg128(hs1, src_a, dst_a, zeros128)
  w2p = jnp.pad(W2, ((0, 0), (0, 16 - _C)))
  ps = _tc2(agg1, hs1, degp, w2p, b1.reshape(1, _D))
  agg2 = _sc_agg16(ps, src_b, dst_b, zeros16)
  b2p = jnp.concatenate(
      [b2, jnp.full((16 - _C,), -1e30, f32)]).reshape(1, 16)
  out = _tc3(agg2, ps, degp, b2p)
  return out[:, :_C]


# speedup vs baseline: 1.0063x; 1.0063x over previous
import functools

import jax
import jax.numpy as jnp
from jax import lax
from jax.experimental import pallas as pl
from jax.experimental.pallas import tpu as pltpu
from jax.experimental.pallas import tpu_sc as plsc

_N = 10000
_E = 320000
_D = 128
_C = 10

_NC = 2
_NS = 16
_NW = _NC * _NS
_EPW = _E // _NW
_NP = 10240
_RPT = _NP // _NS

_sc_mesh = functools.partial(
    plsc.VectorSubcoreMesh, core_axis_name="c", subcore_axis_name="s")
_sc_params = pltpu.CompilerParams(use_tc_tiling_on_sc=False)


def _make_sc_agg(width, K):
  nchunks = _EPW // K

  @functools.partial(
      pl.kernel,
      mesh=_sc_mesh(),
      out_type=jax.ShapeDtypeStruct((_NC, _NP, width), jnp.float32),
      compiler_params=_sc_params,
      scratch_types=[
          pltpu.VMEM((nchunks, K), jnp.int32),
          pltpu.VMEM((nchunks, K), jnp.int32),
          pltpu.VMEM((K, width), jnp.float32),
          pltpu.VMEM((K, width), jnp.float32),
          pltpu.VMEM_SHARED((_NP, width), jnp.float32),
          pltpu.SemaphoreType.DMA,
          pltpu.SemaphoreType.DMA,
          pltpu.SemaphoreType.DMA,
          pltpu.SemaphoreType.DMA,
          pltpu.SemaphoreType.DMA,
      ],
  )
  def agg(table_hbm, src_hbm, dst_hbm, zeros_hbm, out_hbm,
          srcb, dstb, rows0, rows1, acc, sem0, sem1, sem2, sem3, semi):
    c = lax.axis_index("c")
    s = lax.axis_index("s")
    wid = s * _NC + c
    idx_load = pltpu.async_copy(src_hbm.at[wid], srcb, semi)
    idx_load2 = pltpu.async_copy(dst_hbm.at[wid], dstb, semi)
    pltpu.sync_copy(zeros_hbm, acc.at[pl.ds(s * _RPT, _RPT)])
    idx_load.wait()
    idx_load2.wait()
    plsc.subcore_barrier()

    rows = (rows0, rows1)
    gsems = (sem0, sem1)
    ssems = (sem2, sem3)
    gd = [None, None]
    sd = [None, None]
    gd[0] = pltpu.async_copy(table_hbm.at[srcb.at[0]], rows0, gsems[0])
    for i in range(nchunks):
      cur = i % 2
      nxt = 1 - cur
      if i + 1 < nchunks:
        if sd[nxt] is not None:
          sd[nxt].wait()
        gd[nxt] = pltpu.async_copy(table_hbm.at[srcb.at[i + 1]], rows[nxt],
                                   gsems[nxt])
      gd[cur].wait()
      sd[cur] = pltpu.async_copy(rows[cur], acc.at[dstb.at[i]], ssems[cur],
                                 add=True)
    sd[(nchunks - 1) % 2].wait()
    if nchunks > 1:
      sd[nchunks % 2].wait()
    plsc.subcore_barrier()
    pltpu.sync_copy(acc.at[pl.ds(s * _RPT, _RPT)],
                    out_hbm.at[c, pl.ds(s * _RPT, _RPT)])

  return agg


def _make_sc_deg(K):
  nchunks = _EPW // K

  @functools.partial(
      pl.kernel,
      mesh=_sc_mesh(),
      out_type=jax.ShapeDtypeStruct((_NC, _NP, 16), jnp.float32),
      compiler_params=_sc_params,
      scratch_types=[
          pltpu.VMEM((nchunks, K), jnp.int32),
          pltpu.VMEM((K, 16), jnp.float32),
          pltpu.VMEM_SHARED((_NP, 16), jnp.float32),
          pltpu.SemaphoreType.DMA,
      ],
  )
  def deg(dst_hbm, ones_hbm, zeros_hbm, out_hbm, dstb, ones_v, acc, semi):
    c = lax.axis_index("c")
    s = lax.axis_index("s")
    wid = s * _NC + c
    idx_load = pltpu.async_copy(dst_hbm.at[wid], dstb, semi)
    pltpu.sync_copy(zeros_hbm, acc.at[pl.ds(s * _RPT, _RPT)])
    pltpu.sync_copy(ones_hbm, ones_v)
    idx_load.wait()
    plsc.subcore_barrier()

    for i in range(nchunks):
      pltpu.sync_copy(ones_v, acc.at[dstb.at[i]], add=True)
    plsc.subcore_barrier()
    pltpu.sync_copy(acc.at[pl.ds(s * _RPT, _RPT)],
                    out_hbm.at[c, pl.ds(s * _RPT, _RPT)])

  return deg


_BLK = 1000
_GRID = _N // _BLK


def _dinv_of(degp_blk):
  deg = degp_blk[0, :, 0:1] + degp_blk[1, :, 0:1] + 1.0
  return lax.rsqrt(deg)


def _tc1_body(x_ref, w1_ref, degp_ref, hs_ref):
  dinv = _dinv_of(degp_ref[...])
  h = jnp.dot(x_ref[...], w1_ref[...], preferred_element_type=jnp.float32)
  hs_ref[...] = h * dinv


@jax.jit
def _tc1(x, w1, degp):
  return pl.pallas_call(
      _tc1_body,
      grid=(_GRID,),
      in_specs=[
          pl.BlockSpec((_BLK, _D), lambda i: (i, 0)),
          pl.BlockSpec((_D, _D), lambda i: (0, 0)),
          pl.BlockSpec((_NC, _BLK, 16), lambda i: (0, i, 0)),
      ],
      out_specs=pl.BlockSpec((_BLK, _D), lambda i: (i, 0)),
      out_shape=jax.ShapeDtypeStruct((_N, _D), jnp.float32),
  )(x, w1, degp)


def _tc2_body(agg_ref, hs_ref, degp_ref, w2_ref, b1_ref, ps_ref):
  dinv = _dinv_of(degp_ref[...])
  agg = agg_ref[0] + agg_ref[1] + hs_ref[...]
  h = jnp.maximum(agg * dinv + b1_ref[...], 0.0)
  p = jnp.dot(h, w2_ref[...], preferred_element_type=jnp.float32)
  ps_ref[...] = p * dinv


@jax.jit
def _tc2(agg1, hs, degp, w2p, b1):
  return pl.pallas_call(
      _tc2_body,
      grid=(_GRID,),
      in_specs=[
          pl.BlockSpec((_NC, _BLK, _D), lambda i: (0, i, 0)),
          pl.BlockSpec((_BLK, _D), lambda i: (i, 0)),
          pl.BlockSpec((_NC, _BLK, 16), lambda i: (0, i, 0)),
          pl.BlockSpec((_D, 16), lambda i: (0, 0)),
          pl.BlockSpec((1, _D), lambda i: (0, 0)),
      ],
      out_specs=pl.BlockSpec((_BLK, 16), lambda i: (i, 0)),
      out_shape=jax.ShapeDtypeStruct((_N, 16), jnp.float32),
  )(agg1, hs, degp, w2p, b1)


def _tc3_body(agg_ref, ps_ref, degp_ref, b2_ref, out_ref):
  dinv = _dinv_of(degp_ref[...])
  z = (agg_ref[0] + agg_ref[1] + ps_ref[...]) * dinv + b2_ref[...]
  m = jnp.max(z, axis=1, keepdims=True)
  e = jnp.exp(z - m)
  out_ref[...] = e / jnp.sum(e, axis=1, keepdims=True)


@jax.jit
def _tc3(agg2, ps, degp, b2p):
  return pl.pallas_call(
      _tc3_body,
      grid=(_GRID,),
      in_specs=[
          pl.BlockSpec((_NC, _BLK, 16), lambda i: (0, i, 0)),
          pl.BlockSpec((_BLK, 16), lambda i: (i, 0)),
          pl.BlockSpec((_NC, _BLK, 16), lambda i: (0, i, 0)),
          pl.BlockSpec((1, 16), lambda i: (0, 0)),
      ],
      out_specs=pl.BlockSpec((_BLK, 16), lambda i: (i, 0)),
      out_shape=jax.ShapeDtypeStruct((_N, 16), jnp.float32),
  )(agg2, ps, degp, b2p)


_sc_deg = _make_sc_deg(K=2000)
_sc_agg128 = _make_sc_agg(width=_D, K=80)
_sc_agg16 = _make_sc_agg(width=16, K=2000)


@jax.jit
def kernel(node_features, edge_index, W1, b1, W2, b2):
  src = edge_index[0]
  dst = edge_index[1]
  src_a = src.reshape(_NW, _EPW // 80, 80)
  dst_a = dst.reshape(_NW, _EPW // 80, 80)
  src_b = src.reshape(_NW, _EPW // 2000, 2000)
  dst_b = dst.reshape(_NW, _EPW // 2000, 2000)
  f32 = jnp.float32
  zeros16 = jnp.zeros((_RPT, 16), f32)
  zeros128 = jnp.zeros((_RPT, _D), f32)
  ones_rows = jnp.ones((2000, 16), f32)

  degp = _sc_deg(dst_b, ones_rows, zeros16)
  hs1 = _tc1(node_features, W1, degp)
  agg1 = _sc_agg128(hs1, src_a, dst_a, zeros128)
  w2p = jnp.pad(W2, ((0, 0), (0, 16 - _C)))
  ps = _tc2(agg1, hs1, degp, w2p, b1.reshape(1, _D))
  agg2 = _sc_agg16(ps, src_b, dst_b, zeros16)
  b2p = jnp.concatenate(
      [b2, jnp.full((16 - _C,), -1e30, f32)]).reshape(1, 16)
  out = _tc3(agg2, ps, degp, b2p)
  return out[:, :_C]


# speedup vs baseline: 1.0867x; 1.0799x over previous
import functools

import jax
import jax.numpy as jnp
from jax import lax
from jax.experimental import pallas as pl
from jax.experimental.pallas import tpu as pltpu
from jax.experimental.pallas import tpu_sc as plsc

_N = 10000
_E = 320000
_D = 128
_C = 10

_NC = 2
_NS = 16
_NW = _NC * _NS
_EPW = _E // _NW
_NP = 10240
_RPT = _NP // _NS

_sc_mesh = functools.partial(
    plsc.VectorSubcoreMesh, core_axis_name="c", subcore_axis_name="s")
_sc_params = pltpu.CompilerParams(use_tc_tiling_on_sc=False)


def _make_sc_agg(width, K):
  nchunks = _EPW // K

  @functools.partial(
      pl.kernel,
      mesh=_sc_mesh(),
      out_type=jax.ShapeDtypeStruct((_NC, _NP, width), jnp.float32),
      compiler_params=_sc_params,
      scratch_types=[
          pltpu.VMEM((nchunks, K), jnp.int32),
          pltpu.VMEM((nchunks, K), jnp.int32),
          pltpu.VMEM((K, width), jnp.float32),
          pltpu.VMEM((K, width), jnp.float32),
          pltpu.VMEM_SHARED((_NP, width), jnp.float32),
          pltpu.SemaphoreType.DMA,
          pltpu.SemaphoreType.DMA,
          pltpu.SemaphoreType.DMA,
          pltpu.SemaphoreType.DMA,
          pltpu.SemaphoreType.DMA,
      ],
  )
  def agg(table_hbm, edges_hbm, zeros_hbm, out_hbm,
          srcb, dstb, rows0, rows1, acc, sem0, sem1, sem2, sem3, semi):
    c = lax.axis_index("c")
    s = lax.axis_index("s")
    wid = s * _NC + c
    idx_load = pltpu.async_copy(edges_hbm.at[0, wid], srcb, semi)
    idx_load2 = pltpu.async_copy(edges_hbm.at[1, wid], dstb, semi)
    pltpu.sync_copy(zeros_hbm, acc.at[pl.ds(s * _RPT, _RPT)])
    idx_load.wait()
    idx_load2.wait()
    plsc.subcore_barrier()

    rows = (rows0, rows1)
    gsems = (sem0, sem1)
    ssems = (sem2, sem3)
    gd = [None, None]
    sd = [None, None]
    gd[0] = pltpu.async_copy(table_hbm.at[srcb.at[0]], rows0, gsems[0])
    for i in range(nchunks):
      cur = i % 2
      nxt = 1 - cur
      if i + 1 < nchunks:
        if sd[nxt] is not None:
          sd[nxt].wait()
        gd[nxt] = pltpu.async_copy(table_hbm.at[srcb.at[i + 1]], rows[nxt],
                                   gsems[nxt])
      gd[cur].wait()
      sd[cur] = pltpu.async_copy(rows[cur], acc.at[dstb.at[i]], ssems[cur],
                                 add=True)
    sd[(nchunks - 1) % 2].wait()
    if nchunks > 1:
      sd[nchunks % 2].wait()
    plsc.subcore_barrier()
    pltpu.sync_copy(acc.at[pl.ds(s * _RPT, _RPT)],
                    out_hbm.at[c, pl.ds(s * _RPT, _RPT)])

  return agg


def _make_sc_deg(K):
  nchunks = _EPW // K

  @functools.partial(
      pl.kernel,
      mesh=_sc_mesh(),
      out_type=jax.ShapeDtypeStruct((_NC, _NP, 16), jnp.float32),
      compiler_params=_sc_params,
      scratch_types=[
          pltpu.VMEM((nchunks, K), jnp.int32),
          pltpu.VMEM((K, 16), jnp.float32),
          pltpu.VMEM_SHARED((_NP, 16), jnp.float32),
          pltpu.SemaphoreType.DMA,
      ],
  )
  def deg(edges_hbm, ones_hbm, zeros_hbm, out_hbm, dstb, ones_v, acc, semi):
    c = lax.axis_index("c")
    s = lax.axis_index("s")
    wid = s * _NC + c
    idx_load = pltpu.async_copy(edges_hbm.at[1, wid], dstb, semi)
    pltpu.sync_copy(zeros_hbm, acc.at[pl.ds(s * _RPT, _RPT)])
    pltpu.sync_copy(ones_hbm, ones_v)
    idx_load.wait()
    plsc.subcore_barrier()

    for i in range(nchunks):
      pltpu.sync_copy(ones_v, acc.at[dstb.at[i]], add=True)
    plsc.subcore_barrier()
    pltpu.sync_copy(acc.at[pl.ds(s * _RPT, _RPT)],
                    out_hbm.at[c, pl.ds(s * _RPT, _RPT)])

  return deg


_BLK = 2000
_GRID = _N // _BLK


def _dinv_of(degp_blk):
  deg = degp_blk[0, :, 0:1] + degp_blk[1, :, 0:1] + 1.0
  return lax.rsqrt(deg)


def _tc1_body(x_ref, w1_ref, degp_ref, hs_ref):
  dinv = _dinv_of(degp_ref[...])
  h = jnp.dot(x_ref[...], w1_ref[...], preferred_element_type=jnp.float32)
  hs_ref[...] = h * dinv


@jax.jit
def _tc1(x, w1, degp):
  return pl.pallas_call(
      _tc1_body,
      grid=(_GRID,),
      in_specs=[
          pl.BlockSpec((_BLK, _D), lambda i: (i, 0)),
          pl.BlockSpec((_D, _D), lambda i: (0, 0)),
          pl.BlockSpec((_NC, _BLK, 16), lambda i: (0, i, 0)),
      ],
      out_specs=pl.BlockSpec((_BLK, _D), lambda i: (i, 0)),
      out_shape=jax.ShapeDtypeStruct((_N, _D), jnp.float32),
  )(x, w1, degp)


def _tc2_body(agg_ref, hs_ref, degp_ref, w2_ref, b1_ref, ps_ref):
  dinv = _dinv_of(degp_ref[...])
  agg = agg_ref[0] + agg_ref[1] + hs_ref[...]
  h = jnp.maximum(agg * dinv + b1_ref[...], 0.0)
  p = jnp.dot(h, w2_ref[...], preferred_element_type=jnp.float32)
  ps_ref[...] = p * dinv


@jax.jit
def _tc2(agg1, hs, degp, w2p, b1):
  return pl.pallas_call(
      _tc2_body,
      grid=(_GRID,),
      in_specs=[
          pl.BlockSpec((_NC, _BLK, _D), lambda i: (0, i, 0)),
          pl.BlockSpec((_BLK, _D), lambda i: (i, 0)),
          pl.BlockSpec((_NC, _BLK, 16), lambda i: (0, i, 0)),
          pl.BlockSpec((_D, 16), lambda i: (0, 0)),
          pl.BlockSpec((1, _D), lambda i: (0, 0)),
      ],
      out_specs=pl.BlockSpec((_BLK, 16), lambda i: (i, 0)),
      out_shape=jax.ShapeDtypeStruct((_N, 16), jnp.float32),
  )(agg1, hs, degp, w2p, b1)


def _tc3_body(agg_ref, ps_ref, degp_ref, b2_ref, out_ref):
  dinv = _dinv_of(degp_ref[...])
  z = (agg_ref[0] + agg_ref[1] + ps_ref[...]) * dinv + b2_ref[...]
  m = jnp.max(z, axis=1, keepdims=True)
  e = jnp.exp(z - m)
  out_ref[...] = (e / jnp.sum(e, axis=1, keepdims=True))[:, :_C]


@jax.jit
def _tc3(agg2, ps, degp, b2p):
  return pl.pallas_call(
      _tc3_body,
      grid=(_GRID,),
      in_specs=[
          pl.BlockSpec((_NC, _BLK, 16), lambda i: (0, i, 0)),
          pl.BlockSpec((_BLK, 16), lambda i: (i, 0)),
          pl.BlockSpec((_NC, _BLK, 16), lambda i: (0, i, 0)),
          pl.BlockSpec((1, 16), lambda i: (0, 0)),
      ],
      out_specs=pl.BlockSpec((_BLK, _C), lambda i: (i, 0)),
      out_shape=jax.ShapeDtypeStruct((_N, _C), jnp.float32),
  )(agg2, ps, degp, b2p)


_sc_deg = _make_sc_deg(K=2000)
_sc_agg128 = _make_sc_agg(width=_D, K=80)
_sc_agg16 = _make_sc_agg(width=16, K=2000)


@jax.jit
def kernel(node_features, edge_index, W1, b1, W2, b2):
  e4a = edge_index.reshape(2, _NW, _EPW // 80, 80)
  e4b = edge_index.reshape(2, _NW, _EPW // 2000, 2000)
  f32 = jnp.float32
  zeros16 = jnp.zeros((_RPT, 16), f32)
  zeros128 = jnp.zeros((_RPT, _D), f32)
  ones_rows = jnp.ones((2000, 16), f32)

  degp = _sc_deg(e4b, ones_rows, zeros16)
  hs1 = _tc1(node_features, W1, degp)
  agg1 = _sc_agg128(hs1, e4a, zeros128)
  w2p = jnp.pad(W2, ((0, 0), (0, 16 - _C)))
  ps = _tc2(agg1, hs1, degp, w2p, b1.reshape(1, _D))
  agg2 = _sc_agg16(ps, e4b, zeros16)
  b2p = jnp.concatenate(
      [b2, jnp.full((16 - _C,), -1e30, f32)]).reshape(1, 16)
  return _tc3(agg2, ps, degp, b2p)


# speedup vs baseline: 1.0874x; 1.0006x over previous
import functools

import jax
import jax.numpy as jnp
from jax import lax
from jax.experimental import pallas as pl
from jax.experimental.pallas import tpu as pltpu
from jax.experimental.pallas import tpu_sc as plsc

_N = 10000
_E = 320000
_D = 128
_C = 10

_NC = 2
_NS = 16
_NW = _NC * _NS
_EPW = _E // _NW
_NP = 10240
_RPT = _NP // _NS

_sc_mesh = functools.partial(
    plsc.VectorSubcoreMesh, core_axis_name="c", subcore_axis_name="s")
_sc_params = pltpu.CompilerParams(use_tc_tiling_on_sc=False)


def _make_sc_agg(width, K):
  nchunks = _EPW // K

  @functools.partial(
      pl.kernel,
      mesh=_sc_mesh(),
      out_type=jax.ShapeDtypeStruct((_NC, _NP, width), jnp.float32),
      compiler_params=_sc_params,
      scratch_types=[
          pltpu.VMEM((_EPW,), jnp.int32),
          pltpu.VMEM((_EPW,), jnp.int32),
          pltpu.VMEM((K, width), jnp.float32),
          pltpu.VMEM((K, width), jnp.float32),
          pltpu.VMEM_SHARED((_NP, width), jnp.float32),
          pltpu.SemaphoreType.DMA,
          pltpu.SemaphoreType.DMA,
          pltpu.SemaphoreType.DMA,
          pltpu.SemaphoreType.DMA,
          pltpu.SemaphoreType.DMA,
      ],
  )
  def agg(table_hbm, edges_hbm, zeros_hbm, out_hbm,
          srcb, dstb, rows0, rows1, acc, sem0, sem1, sem2, sem3, semi):
    c = lax.axis_index("c")
    s = lax.axis_index("s")
    wid = s * _NC + c
    base = wid * _EPW
    idx_load = pltpu.async_copy(edges_hbm.at[0, pl.ds(base, _EPW)], srcb, semi)
    idx_load2 = pltpu.async_copy(edges_hbm.at[1, pl.ds(base, _EPW)], dstb, semi)
    pltpu.sync_copy(zeros_hbm, acc.at[pl.ds(s * _RPT, _RPT)])
    idx_load.wait()
    idx_load2.wait()
    plsc.subcore_barrier()

    rows = (rows0, rows1)
    gsems = (sem0, sem1)
    ssems = (sem2, sem3)
    gd = [None, None]
    sd = [None, None]
    gd[0] = pltpu.async_copy(table_hbm.at[srcb.at[pl.ds(0, K)]], rows0, gsems[0])
    for i in range(nchunks):
      cur = i % 2
      nxt = 1 - cur
      if i + 1 < nchunks:
        if sd[nxt] is not None:
          sd[nxt].wait()
        gd[nxt] = pltpu.async_copy(
            table_hbm.at[srcb.at[pl.ds((i + 1) * K, K)]], rows[nxt],
            gsems[nxt])
      gd[cur].wait()
      sd[cur] = pltpu.async_copy(rows[cur], acc.at[dstb.at[pl.ds(i * K, K)]],
                                 ssems[cur], add=True)
    sd[(nchunks - 1) % 2].wait()
    if nchunks > 1:
      sd[nchunks % 2].wait()
    plsc.subcore_barrier()
    pltpu.sync_copy(acc.at[pl.ds(s * _RPT, _RPT)],
                    out_hbm.at[c, pl.ds(s * _RPT, _RPT)])

  return agg


def _make_sc_deg(K):
  nchunks = _EPW // K

  @functools.partial(
      pl.kernel,
      mesh=_sc_mesh(),
      out_type=jax.ShapeDtypeStruct((_NC, _NP, 16), jnp.float32),
      compiler_params=_sc_params,
      scratch_types=[
          pltpu.VMEM((_EPW,), jnp.int32),
          pltpu.VMEM((K, 16), jnp.float32),
          pltpu.VMEM_SHARED((_NP, 16), jnp.float32),
          pltpu.SemaphoreType.DMA,
      ],
  )
  def deg(edges_hbm, ones_hbm, zeros_hbm, out_hbm, dstb, ones_v, acc, semi):
    c = lax.axis_index("c")
    s = lax.axis_index("s")
    wid = s * _NC + c
    idx_load = pltpu.async_copy(
        edges_hbm.at[1, pl.ds(wid * _EPW, _EPW)], dstb, semi)
    pltpu.sync_copy(zeros_hbm, acc.at[pl.ds(s * _RPT, _RPT)])
    pltpu.sync_copy(ones_hbm, ones_v)
    idx_load.wait()
    plsc.subcore_barrier()

    for i in range(nchunks):
      pltpu.sync_copy(ones_v, acc.at[dstb.at[pl.ds(i * K, K)]], add=True)
    plsc.subcore_barrier()
    pltpu.sync_copy(acc.at[pl.ds(s * _RPT, _RPT)],
                    out_hbm.at[c, pl.ds(s * _RPT, _RPT)])

  return deg


_BLK = 2000
_GRID = _N // _BLK


def _dinv_of(degp_blk):
  deg = degp_blk[0, :, 0:1] + degp_blk[1, :, 0:1] + 1.0
  return lax.rsqrt(deg)


def _tc1_body(x_ref, w1_ref, degp_ref, hs_ref):
  dinv = _dinv_of(degp_ref[...])
  h = jnp.dot(x_ref[...], w1_ref[...], preferred_element_type=jnp.float32)
  hs_ref[...] = h * dinv


@jax.jit
def _tc1(x, w1, degp):
  return pl.pallas_call(
      _tc1_body,
      grid=(_GRID,),
      in_specs=[
          pl.BlockSpec((_BLK, _D), lambda i: (i, 0)),
          pl.BlockSpec((_D, _D), lambda i: (0, 0)),
          pl.BlockSpec((_NC, _BLK, 16), lambda i: (0, i, 0)),
      ],
      out_specs=pl.BlockSpec((_BLK, _D), lambda i: (i, 0)),
      out_shape=jax.ShapeDtypeStruct((_N, _D), jnp.float32),
  )(x, w1, degp)


def _tc2_body(agg_ref, hs_ref, degp_ref, w2_ref, b1_ref, ps_ref):
  dinv = _dinv_of(degp_ref[...])
  agg = agg_ref[0] + agg_ref[1] + hs_ref[...]
  h = jnp.maximum(agg * dinv + b1_ref[...], 0.0)
  p = jnp.dot(h, w2_ref[...], preferred_element_type=jnp.float32)
  ps_ref[...] = p * dinv


@jax.jit
def _tc2(agg1, hs, degp, w2p, b1):
  return pl.pallas_call(
      _tc2_body,
      grid=(_GRID,),
      in_specs=[
          pl.BlockSpec((_NC, _BLK, _D), lambda i: (0, i, 0)),
          pl.BlockSpec((_BLK, _D), lambda i: (i, 0)),
          pl.BlockSpec((_NC, _BLK, 16), lambda i: (0, i, 0)),
          pl.BlockSpec((_D, 16), lambda i: (0, 0)),
          pl.BlockSpec((1, _D), lambda i: (0, 0)),
      ],
      out_specs=pl.BlockSpec((_BLK, 16), lambda i: (i, 0)),
      out_shape=jax.ShapeDtypeStruct((_N, 16), jnp.float32),
  )(agg1, hs, degp, w2p, b1)


def _tc3_body(agg_ref, ps_ref, degp_ref, b2_ref, out_ref):
  dinv = _dinv_of(degp_ref[...])
  z = (agg_ref[0] + agg_ref[1] + ps_ref[...]) * dinv + b2_ref[...]
  m = jnp.max(z, axis=1, keepdims=True)
  e = jnp.exp(z - m)
  out_ref[...] = (e / jnp.sum(e, axis=1, keepdims=True))[:, :_C]


@jax.jit
def _tc3(agg2, ps, degp, b2p):
  return pl.pallas_call(
      _tc3_body,
      grid=(_GRID,),
      in_specs=[
          pl.BlockSpec((_NC, _BLK, 16), lambda i: (0, i, 0)),
          pl.BlockSpec((_BLK, 16), lambda i: (i, 0)),
          pl.BlockSpec((_NC, _BLK, 16), lambda i: (0, i, 0)),
          pl.BlockSpec((1, 16), lambda i: (0, 0)),
      ],
      out_specs=pl.BlockSpec((_BLK, _C), lambda i: (i, 0)),
      out_shape=jax.ShapeDtypeStruct((_N, _C), jnp.float32),
  )(agg2, ps, degp, b2p)


_sc_deg = _make_sc_deg(K=2000)
_sc_agg128 = _make_sc_agg(width=_D, K=80)
_sc_agg16 = _make_sc_agg(width=16, K=2000)


@jax.jit
def kernel(node_features, edge_index, W1, b1, W2, b2):
  f32 = jnp.float32
  zeros16 = jnp.zeros((_RPT, 16), f32)
  zeros128 = jnp.zeros((_RPT, _D), f32)
  ones_rows = jnp.ones((2000, 16), f32)

  degp = _sc_deg(edge_index, ones_rows, zeros16)
  hs1 = _tc1(node_features, W1, degp)
  agg1 = _sc_agg128(hs1, edge_index, zeros128)
  w2p = jnp.pad(W2, ((0, 0), (0, 16 - _C)))
  ps = _tc2(agg1, hs1, degp, w2p, b1.reshape(1, _D))
  agg2 = _sc_agg16(ps, edge_index, zeros16)
  b2p = jnp.concatenate(
      [b2, jnp.full((16 - _C,), -1e30, f32)]).reshape(1, 16)
  return _tc3(agg2, ps, degp, b2p)


# speedup vs baseline: 1.2698x; 1.1678x over previous
import functools

import jax
import jax.numpy as jnp
from jax import lax
from jax.experimental import pallas as pl
from jax.experimental.pallas import tpu as pltpu
from jax.experimental.pallas import tpu_sc as plsc

_N = 10000
_E = 320000
_D = 128
_C = 10

_NC = 2
_NS = 16
_NW = _NC * _NS
_EPW = _E // _NW
_NP = 10240
_RPT = _NP // _NS

_sc_mesh = functools.partial(
    plsc.VectorSubcoreMesh, core_axis_name="c", subcore_axis_name="s")
_sc_params = pltpu.CompilerParams(use_tc_tiling_on_sc=False)


def _make_sc_agg(width, K, dtype):
  nchunks = _EPW // K

  @functools.partial(
      pl.kernel,
      mesh=_sc_mesh(),
      out_type=jax.ShapeDtypeStruct((_NC, _NP, width), dtype),
      compiler_params=_sc_params,
      scratch_types=[
          pltpu.VMEM((_EPW,), jnp.int32),
          pltpu.VMEM((_EPW,), jnp.int32),
          pltpu.VMEM((K, width), dtype),
          pltpu.VMEM((K, width), dtype),
          pltpu.VMEM_SHARED((_NP, width), dtype),
          pltpu.SemaphoreType.DMA,
          pltpu.SemaphoreType.DMA,
          pltpu.SemaphoreType.DMA,
          pltpu.SemaphoreType.DMA,
          pltpu.SemaphoreType.DMA,
      ],
  )
  def agg(table_hbm, edges_hbm, zeros_hbm, out_hbm,
          srcb, dstb, rows0, rows1, acc, sem0, sem1, sem2, sem3, semi):
    c = lax.axis_index("c")
    s = lax.axis_index("s")
    wid = s * _NC + c
    base = wid * _EPW
    idx_load = pltpu.async_copy(edges_hbm.at[0, pl.ds(base, _EPW)], srcb, semi)
    idx_load2 = pltpu.async_copy(edges_hbm.at[1, pl.ds(base, _EPW)], dstb, semi)
    pltpu.sync_copy(zeros_hbm, acc.at[pl.ds(s * _RPT, _RPT)])
    idx_load.wait()
    idx_load2.wait()
    plsc.subcore_barrier()

    rows = (rows0, rows1)
    gsems = (sem0, sem1)
    ssems = (sem2, sem3)
    gd = [None, None]
    sd = [None, None]
    gd[0] = pltpu.async_copy(table_hbm.at[srcb.at[pl.ds(0, K)]], rows0, gsems[0])
    for i in range(nchunks):
      cur = i % 2
      nxt = 1 - cur
      if i + 1 < nchunks:
        if sd[nxt] is not None:
          sd[nxt].wait()
        gd[nxt] = pltpu.async_copy(
            table_hbm.at[srcb.at[pl.ds((i + 1) * K, K)]], rows[nxt],
            gsems[nxt])
      gd[cur].wait()
      sd[cur] = pltpu.async_copy(rows[cur], acc.at[dstb.at[pl.ds(i * K, K)]],
                                 ssems[cur], add=True)
    sd[(nchunks - 1) % 2].wait()
    if nchunks > 1:
      sd[nchunks % 2].wait()
    plsc.subcore_barrier()
    pltpu.sync_copy(acc.at[pl.ds(s * _RPT, _RPT)],
                    out_hbm.at[c, pl.ds(s * _RPT, _RPT)])

  return agg


def _make_sc_deg(K):
  nchunks = _EPW // K

  @functools.partial(
      pl.kernel,
      mesh=_sc_mesh(),
      out_type=jax.ShapeDtypeStruct((_NC, _NP, 16), jnp.float32),
      compiler_params=_sc_params,
      scratch_types=[
          pltpu.VMEM((_EPW,), jnp.int32),
          pltpu.VMEM((K, 16), jnp.float32),
          pltpu.VMEM_SHARED((_NP, 16), jnp.float32),
          pltpu.SemaphoreType.DMA,
      ],
  )
  def deg(edges_hbm, ones_hbm, zeros_hbm, out_hbm, dstb, ones_v, acc, semi):
    c = lax.axis_index("c")
    s = lax.axis_index("s")
    wid = s * _NC + c
    idx_load = pltpu.async_copy(
        edges_hbm.at[1, pl.ds(wid * _EPW, _EPW)], dstb, semi)
    pltpu.sync_copy(zeros_hbm, acc.at[pl.ds(s * _RPT, _RPT)])
    pltpu.sync_copy(ones_hbm, ones_v)
    idx_load.wait()
    plsc.subcore_barrier()

    for i in range(nchunks):
      pltpu.sync_copy(ones_v, acc.at[dstb.at[pl.ds(i * K, K)]], add=True)
    plsc.subcore_barrier()
    pltpu.sync_copy(acc.at[pl.ds(s * _RPT, _RPT)],
                    out_hbm.at[c, pl.ds(s * _RPT, _RPT)])

  return deg


_BLK = 2000
_GRID = _N // _BLK


def _dinv_of(degp_blk):
  deg = degp_blk[0, :, 0:1] + degp_blk[1, :, 0:1] + 1.0
  return lax.rsqrt(deg)


def _tc1_body(x_ref, w1_ref, degp_ref, hs_ref):
  dinv = _dinv_of(degp_ref[...])
  h = jnp.dot(x_ref[...], w1_ref[...], preferred_element_type=jnp.float32)
  hs_ref[...] = (h * dinv).astype(jnp.bfloat16)


@jax.jit
def _tc1(x, w1, degp):
  return pl.pallas_call(
      _tc1_body,
      grid=(_GRID,),
      in_specs=[
          pl.BlockSpec((_BLK, _D), lambda i: (i, 0)),
          pl.BlockSpec((_D, _D), lambda i: (0, 0)),
          pl.BlockSpec((_NC, _BLK, 16), lambda i: (0, i, 0)),
      ],
      out_specs=pl.BlockSpec((_BLK, _D), lambda i: (i, 0)),
      out_shape=jax.ShapeDtypeStruct((_N, _D), jnp.bfloat16),
  )(x, w1, degp)


def _tc2_body(agg_ref, hs_ref, degp_ref, w2_ref, b1_ref, ps_ref):
  dinv = _dinv_of(degp_ref[...])
  agg = (agg_ref[0].astype(jnp.float32) + agg_ref[1].astype(jnp.float32)
         + hs_ref[...].astype(jnp.float32))
  h = jnp.maximum(agg * dinv + b1_ref[...], 0.0)
  p = jnp.dot(h, w2_ref[...], preferred_element_type=jnp.float32)
  ps_ref[...] = (p * dinv).astype(jnp.bfloat16)


@jax.jit
def _tc2(agg1, hs, degp, w2p, b1):
  return pl.pallas_call(
      _tc2_body,
      grid=(_GRID,),
      in_specs=[
          pl.BlockSpec((_NC, _BLK, _D), lambda i: (0, i, 0)),
          pl.BlockSpec((_BLK, _D), lambda i: (i, 0)),
          pl.BlockSpec((_NC, _BLK, 16), lambda i: (0, i, 0)),
          pl.BlockSpec((_D, 16), lambda i: (0, 0)),
          pl.BlockSpec((1, _D), lambda i: (0, 0)),
      ],
      out_specs=pl.BlockSpec((_BLK, 16), lambda i: (i, 0)),
      out_shape=jax.ShapeDtypeStruct((_N, 16), jnp.bfloat16),
  )(agg1, hs, degp, w2p, b1)


def _tc3_body(agg_ref, ps_ref, degp_ref, b2_ref, out_ref):
  dinv = _dinv_of(degp_ref[...])
  z = (agg_ref[0].astype(jnp.float32) + agg_ref[1].astype(jnp.float32)
       + ps_ref[...].astype(jnp.float32)) * dinv + b2_ref[...]
  m = jnp.max(z, axis=1, keepdims=True)
  e = jnp.exp(z - m)
  out_ref[...] = (e / jnp.sum(e, axis=1, keepdims=True))[:, :_C]


@jax.jit
def _tc3(agg2, ps, degp, b2p):
  return pl.pallas_call(
      _tc3_body,
      grid=(_GRID,),
      in_specs=[
          pl.BlockSpec((_NC, _BLK, 16), lambda i: (0, i, 0)),
          pl.BlockSpec((_BLK, 16), lambda i: (i, 0)),
          pl.BlockSpec((_NC, _BLK, 16), lambda i: (0, i, 0)),
          pl.BlockSpec((1, 16), lambda i: (0, 0)),
      ],
      out_specs=pl.BlockSpec((_BLK, _C), lambda i: (i, 0)),
      out_shape=jax.ShapeDtypeStruct((_N, _C), jnp.float32),
  )(agg2, ps, degp, b2p)


_sc_deg = _make_sc_deg(K=2000)
_sc_agg128 = _make_sc_agg(width=_D, K=400, dtype=jnp.bfloat16)
_sc_agg16 = _make_sc_agg(width=16, K=2000, dtype=jnp.bfloat16)


@jax.jit
def kernel(node_features, edge_index, W1, b1, W2, b2):
  f32 = jnp.float32
  zeros16 = jnp.zeros((_RPT, 16), f32)
  zeros16b = jnp.zeros((_RPT, 16), jnp.bfloat16)
  zeros128b = jnp.zeros((_RPT, _D), jnp.bfloat16)
  ones_rows = jnp.ones((2000, 16), f32)

  degp = _sc_deg(edge_index, ones_rows, zeros16)
  hs1 = _tc1(node_features, W1, degp)
  agg1 = _sc_agg128(hs1, edge_index, zeros128b)
  w2p = jnp.pad(W2, ((0, 0), (0, 16 - _C)))
  ps = _tc2(agg1, hs1, degp, w2p, b1.reshape(1, _D))
  agg2 = _sc_agg16(ps, edge_index, zeros16b)
  b2p = jnp.concatenate(
      [b2, jnp.full((16 - _C,), -1e30, f32)]).reshape(1, 16)
  return _tc3(agg2, ps, degp, b2p)
